# baseline (speedup 1.0000x reference)
import jax
import jax.numpy as jnp
from jax import lax
from jax.experimental import pallas as pl
from jax.experimental.pallas import tpu as pltpu
from jax.experimental.pallas import tpu_sc as plsc

N = 10000
D = 128
B = 16
NTILES = 32
GRP = 128
N_PAD = 10240
ROWS_PER_TILE = N_PAD // 16
RCH = ROWS_PER_TILE // GRP

CH = 8


def _tile_coords():
    cid = lax.axis_index("c")
    sid = lax.axis_index("s")
    return cid, sid, cid * 16 + sid, sid * ROWS_PER_TILE


def _make_sc_agg(G):
    mesh = plsc.VectorSubcoreMesh(core_axis_name="c", subcore_axis_name="s",
                                  num_cores=2, num_subcores=16)

    def kfn(feat, srcg, dstf, z8, parts, sidx, d0, d1, d2, d3, d4, d5, d6,
            d7, rows, accf, sem):
        dj = [d0, d1, d2, d3, d4, d5, d6, d7]
        cid, sid, wid, s0 = _tile_coords()
        ebase = wid * (srcg.shape[1] * GRP)
        pltpu.sync_copy(z8, rows.at[pl.ds(0, 8)])
        for k in range(ROWS_PER_TILE // 8):
            pltpu.sync_copy(rows.at[pl.ds(0, 8)],
                            accf.at[pl.ds(s0 + k * 8, 8)])
        plsc.subcore_barrier()

        def body(c, carry):
            pltpu.sync_copy(srcg.at[wid].at[pl.ds(c * CH, CH)], sidx)
            for j in range(CH):
                pltpu.sync_copy(
                    dstf.at[pl.ds(ebase + (c * CH + j) * GRP, GRP)], dj[j])
            for j in range(CH):
                pltpu.async_copy(feat.at[sidx.at[j]], rows, sem).wait()
                pltpu.sync_copy(rows, accf.at[dj[j]], add=True)
            return carry

        lax.fori_loop(0, G // CH, body, 0)
        plsc.subcore_barrier()

        for k in range(RCH):
            pltpu.sync_copy(accf.at[pl.ds(s0 + k * GRP, GRP)], rows)
            pltpu.sync_copy(rows, parts.at[cid].at[pl.ds(s0 + k * GRP, GRP)])

    return pl.kernel(
        kfn, mesh=mesh,
        out_type=[jax.ShapeDtypeStruct((2, N_PAD, D), jnp.float32)],
        scratch_types=[
            pltpu.VMEM((CH, GRP), jnp.int32),
            *[pltpu.VMEM((GRP,), jnp.int32) for _ in range(CH)],
            pltpu.VMEM((GRP, D), jnp.float32),
            pltpu.VMEM_SHARED((N_PAD, D), jnp.float32),
            pltpu.SemaphoreType.DMA,
        ])


def _dense_body(p0, p1, d0, d1, xb, wl, wr, b, o):
    deg = jnp.maximum(d0[:, 0:1] + d1[:, 0:1], 1.0)
    agg = (p0[...] + p1[...]) / deg
    acc = jnp.dot(agg, wl[...], preferred_element_type=jnp.float32)
    acc += jnp.dot(xb[...], wr[...], preferred_element_type=jnp.float32)
    o[...] = jnp.maximum(acc + b[...], 0.0)


def _dense(p0, p1, d0, d1, x, Wl, Wr, b):
    BLK = 400
    grid = (N // BLK,)
    return pl.pallas_call(
        _dense_body,
        grid=grid,
        in_specs=[
            pl.BlockSpec((BLK, D), lambda i: (i, 0)),
            pl.BlockSpec((BLK, D), lambda i: (i, 0)),
            pl.BlockSpec((BLK, D), lambda i: (i, 0)),
            pl.BlockSpec((BLK, D), lambda i: (i, 0)),
            pl.BlockSpec((BLK, D), lambda i: (i, 0)),
            pl.BlockSpec((D, D), lambda i: (0, 0)),
            pl.BlockSpec((D, D), lambda i: (0, 0)),
            pl.BlockSpec((1, D), lambda i: (0, 0)),
        ],
        out_specs=pl.BlockSpec((BLK, D), lambda i: (i, 0)),
        out_shape=jax.ShapeDtypeStruct((N, D), jnp.float32),
    )(p0, p1, d0, d1, x, Wl, Wr, b.reshape(1, D))


def _pool_body(h, bb, wc1, bc1, wc2r, bc2, o, gsum, gmax, gcnt):
    i = pl.program_id(0)
    nsteps = pl.num_programs(0)

    @pl.when(i == 0)
    def _init():
        gsum[...] = jnp.zeros_like(gsum)
        gcnt[...] = jnp.zeros_like(gcnt)
        gmax[...] = jnp.full_like(gmax, -jnp.inf)

    hb = h[...]
    onehot = (bb[...] == lax.broadcasted_iota(jnp.int32, (1, B), 1)
              ).astype(jnp.float32)
    gsum[...] += lax.dot_general(onehot, hb, (((0,), (0,)), ((), ())),
                                 preferred_element_type=jnp.float32)
    cnt = jnp.sum(onehot, axis=0)
    gcnt[...] += jnp.broadcast_to(cnt[:, None], gcnt.shape)
    neg = jnp.float32(-jnp.inf)
    for bidx in range(B):
        m = onehot[:, bidx:bidx + 1] > 0.0
        mx = jnp.max(jnp.where(m, hb, neg), axis=0)
        gmax[bidx:bidx + 1, :] = jnp.maximum(gmax[bidx:bidx + 1, :],
                                             mx[None, :])

    @pl.when(i == nsteps - 1)
    def _fin():
        mean = gsum[...] / jnp.maximum(gcnt[...], 1.0)
        g = jnp.concatenate([mean, gmax[...]], axis=1)
        z = jnp.dot(g, wc1[...], preferred_element_type=jnp.float32)
        z = jnp.maximum(z + bc1[...], 0.0)
        val = jnp.sum(z * wc2r[...], axis=1, keepdims=True) + bc2[0, 0]
        sig = 1.0 / (1.0 + jnp.exp(-val))
        o[...] = jnp.broadcast_to(sig, o.shape)


def _pool(h, batch2d, Wc1, bc1, wc2r, bc2):
    BLK = 400
    grid = (N // BLK,)
    return pl.pallas_call(
        _pool_body,
        grid=grid,
        in_specs=[
            pl.BlockSpec((BLK, D), lambda i: (i, 0)),
            pl.BlockSpec((BLK, 1), lambda i: (i, 0)),
            pl.BlockSpec((2 * D, D), lambda i: (0, 0)),
            pl.BlockSpec((1, D), lambda i: (0, 0)),
            pl.BlockSpec((1, D), lambda i: (0, 0)),
            pl.BlockSpec(memory_space=pltpu.SMEM),
        ],
        out_specs=pl.BlockSpec((B, D), lambda i: (0, 0)),
        out_shape=jax.ShapeDtypeStruct((B, D), jnp.float32),
        scratch_shapes=[
            pltpu.VMEM((B, D), jnp.float32),
            pltpu.VMEM((B, D), jnp.float32),
            pltpu.VMEM((B, D), jnp.float32),
        ],
    )(h, batch2d, Wc1, bc1, wc2r, bc2)


def kernel(x, edge_index, batch, W1l, W1r, b1, W2l, W2r, b2, Wc1, bc1, Wc2, bc2):
    E = edge_index.shape[1]
    G = -(-E // (NTILES * GRP))
    G = -(-G // CH) * CH
    E_pad = NTILES * G * GRP
    src = edge_index[0].astype(jnp.int32)
    dst = edge_index[1].astype(jnp.int32)
    pad = E_pad - E
    src_p = jnp.concatenate([src, jnp.zeros((pad,), jnp.int32)])
    dst_p = jnp.concatenate([dst, jnp.full((pad,), N, jnp.int32)])
    srcg = src_p.reshape(NTILES, G, GRP)
    dstg = dst_p

    agg = _make_sc_agg(G)

    def _one(r):
        return r[0] if isinstance(r, (list, tuple)) else r

    z8 = jnp.zeros((8, D), jnp.float32)
    onesT = jnp.ones((8, D), jnp.float32)
    srcz = jnp.zeros_like(srcg)

    parts = _one(agg(x, srcg, dstg, z8))
    onesT_dep = onesT + 0.0 * parts[0, 0, 0]
    degp = _one(agg(onesT_dep, srcz, dstg, z8))
    d0 = degp[0, :N]
    d1 = degp[1, :N]
    h1 = _dense(parts[0, :N], parts[1, :N], d0, d1, x, W1l, W1r, b1)
    parts2 = _one(agg(h1, srcg, dstg, z8))
    h2 = _dense(parts2[0, :N], parts2[1, :N], d0, d1, h1, W2l, W2r, b2)

    outp = _pool(h2, batch.astype(jnp.int32).reshape(N, 1),
                 Wc1, bc1.reshape(1, D), Wc2.reshape(1, D),
                 bc2.reshape(1, 1))
    return outp[:, :1]


# speedup vs baseline: 11.2733x; 11.2733x over previous
import jax
import jax.numpy as jnp
from jax import lax
from jax.experimental import pallas as pl
from jax.experimental.pallas import tpu as pltpu
from jax.experimental.pallas import tpu_sc as plsc

N = 10000
D = 128
B = 16
NTILES = 32
GRP = 128
N_PAD = 10240
ROWS_PER_TILE = N_PAD // 16
RCH = ROWS_PER_TILE // GRP

CH = 8


def _tile_coords():
    cid = lax.axis_index("c")
    sid = lax.axis_index("s")
    return cid, sid, cid * 16 + sid, sid * ROWS_PER_TILE


def _make_sc_agg(G, gather_feat=True):
    mesh = plsc.VectorSubcoreMesh(core_axis_name="c", subcore_axis_name="s",
                                  num_cores=2, num_subcores=16)

    def kfn(feat, srcg, dstf, z8, parts, sidx, d0, d1, d2, d3, d4, d5, d6,
            d7, rows0, rows1, accf, sem0, sem1):
        dj = [d0, d1, d2, d3, d4, d5, d6, d7]
        rbuf = [rows0, rows1]
        sems = [sem0, sem1]
        cid, sid, wid, s0 = _tile_coords()
        ebase = wid * (srcg.shape[1] * GRP)
        pltpu.sync_copy(z8, rows0.at[pl.ds(0, 8)])
        for k in range(ROWS_PER_TILE // 8):
            pltpu.sync_copy(rows0.at[pl.ds(0, 8)],
                            accf.at[pl.ds(s0 + k * 8, 8)])
        if not gather_feat:
            for r in range(GRP // 8):
                pltpu.sync_copy(feat, rows0.at[pl.ds(r * 8, 8)])
        plsc.subcore_barrier()

        def body(c, carry):
            if gather_feat:
                pltpu.sync_copy(srcg.at[wid].at[pl.ds(c * CH, CH)], sidx)
            for j in range(CH):
                pltpu.sync_copy(
                    dstf.at[pl.ds(ebase + (c * CH + j) * GRP, GRP)], dj[j])
            if gather_feat:
                cp = pltpu.async_copy(feat.at[sidx.at[0]], rbuf[0], sem0)
                cps = [cp]
                for j in range(CH):
                    if j + 1 < CH:
                        cps.append(pltpu.async_copy(
                            feat.at[sidx.at[j + 1]], rbuf[(j + 1) % 2],
                            sems[(j + 1) % 2]))
                    cps[j].wait()
                    pltpu.sync_copy(rbuf[j % 2], accf.at[dj[j]], add=True)
            else:
                for j in range(CH):
                    pltpu.sync_copy(rows0, accf.at[dj[j]], add=True)
            return carry

        lax.fori_loop(0, G // CH, body, 0)
        plsc.subcore_barrier()

        for k in range(RCH):
            pltpu.sync_copy(accf.at[pl.ds(s0 + k * GRP, GRP)], rows0)
            pltpu.sync_copy(rows0, parts.at[cid].at[pl.ds(s0 + k * GRP, GRP)])

    return pl.kernel(
        kfn, mesh=mesh,
        out_type=[jax.ShapeDtypeStruct((2, N_PAD, D), jnp.float32)],
        scratch_types=[
            pltpu.VMEM((CH, GRP), jnp.int32),
            *[pltpu.VMEM((GRP,), jnp.int32) for _ in range(CH)],
            pltpu.VMEM((GRP, D), jnp.float32),
            pltpu.VMEM((GRP, D), jnp.float32),
            pltpu.VMEM_SHARED((N_PAD, D), jnp.float32),
            pltpu.SemaphoreType.DMA,
            pltpu.SemaphoreType.DMA,
        ])


def _dense_body(p0, p1, d0, d1, xb, wl, wr, b, o):
    deg = jnp.maximum(d0[:, 0:1] + d1[:, 0:1], 1.0)
    agg = (p0[...] + p1[...]) / deg
    acc = jnp.dot(agg, wl[...], preferred_element_type=jnp.float32)
    acc += jnp.dot(xb[...], wr[...], preferred_element_type=jnp.float32)
    o[...] = jnp.maximum(acc + b[...], 0.0)


def _dense(p0, p1, d0, d1, x, Wl, Wr, b):
    BLK = 400
    grid = (N // BLK,)
    return pl.pallas_call(
        _dense_body,
        grid=grid,
        in_specs=[
            pl.BlockSpec((BLK, D), lambda i: (i, 0)),
            pl.BlockSpec((BLK, D), lambda i: (i, 0)),
            pl.BlockSpec((BLK, D), lambda i: (i, 0)),
            pl.BlockSpec((BLK, D), lambda i: (i, 0)),
            pl.BlockSpec((BLK, D), lambda i: (i, 0)),
            pl.BlockSpec((D, D), lambda i: (0, 0)),
            pl.BlockSpec((D, D), lambda i: (0, 0)),
            pl.BlockSpec((1, D), lambda i: (0, 0)),
        ],
        out_specs=pl.BlockSpec((BLK, D), lambda i: (i, 0)),
        out_shape=jax.ShapeDtypeStruct((N, D), jnp.float32),
    )(p0, p1, d0, d1, x, Wl, Wr, b.reshape(1, D))


def _pool_body(h, bb, wc1, bc1, wc2r, bc2, o, gsum, gmax, gcnt):
    i = pl.program_id(0)
    nsteps = pl.num_programs(0)

    @pl.when(i == 0)
    def _init():
        gsum[...] = jnp.zeros_like(gsum)
        gcnt[...] = jnp.zeros_like(gcnt)
        gmax[...] = jnp.full_like(gmax, -jnp.inf)

    hb = h[...]
    onehot = (bb[...] == lax.broadcasted_iota(jnp.int32, (1, B), 1)
              ).astype(jnp.float32)
    gsum[...] += lax.dot_general(onehot, hb, (((0,), (0,)), ((), ())),
                                 preferred_element_type=jnp.float32)
    cnt = jnp.sum(onehot, axis=0)
    gcnt[...] += jnp.broadcast_to(cnt[:, None], gcnt.shape)
    neg = jnp.float32(-jnp.inf)
    for bidx in range(B):
        m = onehot[:, bidx:bidx + 1] > 0.0
        mx = jnp.max(jnp.where(m, hb, neg), axis=0)
        gmax[bidx:bidx + 1, :] = jnp.maximum(gmax[bidx:bidx + 1, :],
                                             mx[None, :])

    @pl.when(i == nsteps - 1)
    def _fin():
        mean = gsum[...] / jnp.maximum(gcnt[...], 1.0)
        g = jnp.concatenate([mean, gmax[...]], axis=1)
        z = jnp.dot(g, wc1[...], preferred_element_type=jnp.float32)
        z = jnp.maximum(z + bc1[...], 0.0)
        val = jnp.sum(z * wc2r[...], axis=1, keepdims=True) + bc2[0, 0]
        sig = 1.0 / (1.0 + jnp.exp(-val))
        o[...] = jnp.broadcast_to(sig, o.shape)


def _pool(h, batch2d, Wc1, bc1, wc2r, bc2):
    BLK = 400
    grid = (N // BLK,)
    return pl.pallas_call(
        _pool_body,
        grid=grid,
        in_specs=[
            pl.BlockSpec((BLK, D), lambda i: (i, 0)),
            pl.BlockSpec((BLK, 1), lambda i: (i, 0)),
            pl.BlockSpec((2 * D, D), lambda i: (0, 0)),
            pl.BlockSpec((1, D), lambda i: (0, 0)),
            pl.BlockSpec((1, D), lambda i: (0, 0)),
            pl.BlockSpec(memory_space=pltpu.SMEM),
        ],
        out_specs=pl.BlockSpec((B, D), lambda i: (0, 0)),
        out_shape=jax.ShapeDtypeStruct((B, D), jnp.float32),
        scratch_shapes=[
            pltpu.VMEM((B, D), jnp.float32),
            pltpu.VMEM((B, D), jnp.float32),
            pltpu.VMEM((B, D), jnp.float32),
        ],
    )(h, batch2d, Wc1, bc1, wc2r, bc2)


def kernel(x, edge_index, batch, W1l, W1r, b1, W2l, W2r, b2, Wc1, bc1, Wc2, bc2):
    E = edge_index.shape[1]
    G = -(-E // (NTILES * GRP))
    G = -(-G // CH) * CH
    E_pad = NTILES * G * GRP
    src = edge_index[0].astype(jnp.int32)
    dst = edge_index[1].astype(jnp.int32)
    pad = E_pad - E
    src_p = jnp.concatenate([src, jnp.zeros((pad,), jnp.int32)])
    dst_p = jnp.concatenate([dst, jnp.full((pad,), N, jnp.int32)])
    srcg = src_p.reshape(NTILES, G, GRP)
    dstg = dst_p

    agg = _make_sc_agg(G)

    def _one(r):
        return r[0] if isinstance(r, (list, tuple)) else r

    z8 = jnp.zeros((8, D), jnp.float32)
    onesT = jnp.ones((8, D), jnp.float32)
    srcz = jnp.zeros_like(srcg)

    agg_const = _make_sc_agg(G, gather_feat=False)
    parts = _one(agg(x, srcg, dstg, z8))
    onesT_dep = onesT + 0.0 * parts[0, 0, 0]
    degp = _one(agg_const(onesT_dep, srcz, dstg, z8))
    d0 = degp[0, :N]
    d1 = degp[1, :N]
    h1 = _dense(parts[0, :N], parts[1, :N], d0, d1, x, W1l, W1r, b1)
    parts2 = _one(agg(h1, srcg, dstg, z8))
    h2 = _dense(parts2[0, :N], parts2[1, :N], d0, d1, h1, W2l, W2r, b2)

    outp = _pool(h2, batch.astype(jnp.int32).reshape(N, 1),
                 Wc1, bc1.reshape(1, D), Wc2.reshape(1, D),
                 bc2.reshape(1, 1))
    return outp[:, :1]
